# baseline (speedup 1.0000x reference)
import functools

import jax
import jax.numpy as jnp
from jax import lax
from jax.experimental import pallas as pl
from jax.experimental.pallas import tpu as pltpu
from jax.experimental.pallas import tpu_sc as plsc


def _ln_relu_body(x_ref, g_ref, b_ref, h_ref):
    x = x_ref[...]
    mu = jnp.mean(x, axis=-1, keepdims=True)
    xc = x - mu
    var = jnp.mean(xc * xc, axis=-1, keepdims=True)
    h = xc * lax.rsqrt(var + 1e-5) * g_ref[...] + b_ref[...]
    h_ref[...] = jnp.maximum(h, 0.0)


def _ln_relu(x, ln_gamma, ln_beta, bn):
    n, d = x.shape
    return pl.pallas_call(
        _ln_relu_body,
        grid=(n // bn,),
        in_specs=[
            pl.BlockSpec((bn, d), lambda i: (i, 0)),
            pl.BlockSpec((1, d), lambda i: (0, 0)),
            pl.BlockSpec((1, d), lambda i: (0, 0)),
        ],
        out_specs=pl.BlockSpec((bn, d), lambda i: (i, 0)),
        out_shape=jax.ShapeDtypeStruct((n, d), jnp.float32),
    )(x, ln_gamma.reshape(1, d), ln_beta.reshape(1, d))


_NC = 2
_NS = 16
_L = 16
_CB = 64
_PASSES = 2
_B = 80


def _make_edge_kernel(n, e):
    ept = e // _NS
    nblk = ept // _B
    npad = -(-n // (_NS * 8)) * (_NS * 8)
    rpt = npad // _NS

    assert nblk % 4 == 1 and nblk >= 5
    kmax = nblk // 4

    mesh = plsc.VectorSubcoreMesh(core_axis_name="c", subcore_axis_name="s")

    @functools.partial(
        pl.kernel,
        out_type=jax.ShapeDtypeStruct((2 * _PASSES, npad, 2 * _CB),
                                       jnp.float32),
        mesh=mesh,
        scratch_types=[
            pltpu.VMEM((ept,), jnp.int32),
            [pltpu.VMEM((_B,), jnp.int32)] * 2,
            [pltpu.VMEM((_B,), jnp.int32)] * 4,
            [pltpu.VMEM((_B, _CB), jnp.float32)] * 2,
            [pltpu.VMEM((_B, 2 * _CB), jnp.float32)] * 2,
            pltpu.VMEM((_L,), jnp.float32),
            pltpu.VMEM_SHARED((npad, 2 * _CB), jnp.float32),
            [pltpu.SemaphoreType.DMA] * 2,
            [pltpu.SemaphoreType.DMA] * 2,
            [pltpu.SemaphoreType.DMA] * 4,
        ],
        compiler_params=pltpu.CompilerParams(use_tc_tiling_on_sc=False),
    )
    def edge_kernel(h4, ei_flat, tvec, zrows, ss_out,
                    src_v, gidx, sidx, rows, comb, tv,
                    acc, gsem, ssem, dsem):
        core = lax.axis_index("c")
        sid = lax.axis_index("s")
        ebase = sid * ept
        rbase = sid * rpt

        pltpu.sync_copy(ei_flat.at[pl.ds(ebase, ept)], src_v)
        pltpu.sync_copy(tvec, tv)
        tb = tv[...]

        for cc in range(_PASSES):
            chunk = core * _PASSES + cc

            pltpu.sync_copy(zrows, acc.at[pl.ds(rbase, rpt)])
            plsc.subcore_barrier()

            def build_gidx(blk, p):
                e0 = blk * _B
                for j in range(_B // _L):
                    s16 = src_v[pl.ds(e0 + j * _L, _L)]
                    gidx[p][pl.ds(j * _L, _L)] = s16 * 4 + chunk

            def compute(p):
                def edge_body(ee, c2):
                    for u in range(4):
                        for g in range(_CB // _L):
                            r = rows[p][ee * 4 + u, pl.ds(g * _L, _L)]
                            px = r + 1e-7
                            ex = jnp.exp(px * tb)
                            comb[p][ee * 4 + u, pl.ds(g * _L, _L)] = ex
                            comb[p][ee * 4 + u,
                                    pl.ds(_CB + g * _L, _L)] = px * ex
                    return c2
                lax.fori_loop(0, _B // 4, edge_body, 0)

            def issue_gather(blk, p):
                build_gidx(blk, p)
                pltpu.async_copy(h4.at[gidx[p]], rows[p], gsem[p])

            def wait_gather(p):
                pltpu.make_async_copy(h4.at[gidx[p]], rows[p], gsem[p]).wait()

            def issue_scatter(s, p):
                pltpu.async_copy(comb[p], acc.at[sidx[s]], ssem[p], add=True)

            def wait_scatter(s, p):
                pltpu.make_async_copy(comb[p], acc.at[sidx[s]],
                                      ssem[p]).wait()

            def issue_dst(blk, s):
                pltpu.async_copy(
                    ei_flat.at[pl.ds(e + ebase + blk * _B, _B)], sidx[s],
                    dsem[s])

            def wait_dst(blk, s):
                pltpu.make_async_copy(
                    ei_flat.at[pl.ds(e + ebase + blk * _B, _B)], sidx[s],
                    dsem[s]).wait()

            def phase(i, k):
                blk = 4 * k + i
                p = i % 2
                snext = (i + 2) % 4

                wait_gather(p)
                if i >= 2:
                    wait_scatter(snext, p)
                else:
                    @pl.when(k > 0)
                    def _():
                        wait_scatter(snext, p)
                if i == 3:
                    @pl.when(k < kmax - 1)
                    def _():
                        issue_dst(blk + 2, snext)
                else:
                    issue_dst(blk + 2, snext)
                compute(p)
                wait_dst(blk, i)
                issue_scatter(i, p)
                if i == 3:
                    @pl.when(k < kmax - 1)
                    def _():
                        issue_gather(blk + 2, p)
                else:
                    issue_gather(blk + 2, p)

            issue_dst(0, 0)
            issue_dst(1, 1)
            issue_gather(0, 0)
            issue_gather(1, 1)

            def loop_body(k, carry):
                phase(0, k)
                phase(1, k)
                phase(2, k)
                phase(3, k)
                return carry

            lax.fori_loop(0, kmax, loop_body, 0)

            wait_gather(0)
            wait_scatter(2, 0)
            compute(0)
            wait_dst(nblk - 1, 0)
            issue_scatter(0, 0)
            wait_scatter(3, 1)
            wait_scatter(0, 0)
            plsc.subcore_barrier()

            @pl.when(core == 0)
            def _():
                pltpu.sync_copy(acc.at[pl.ds(rbase, rpt)],
                                ss_out.at[cc, pl.ds(rbase, rpt)])

            @pl.when(core == 1)
            def _():
                pltpu.sync_copy(acc.at[pl.ds(rbase, rpt)],
                                ss_out.at[_PASSES + cc, pl.ds(rbase, rpt)])

    return edge_kernel


def _mlp_body(x_ref, h_ref, ss_ref, w1_ref, b1_ref, bng_ref,
              bnb_ref, w2_ref, b2_ref, o_ref):
    h = h_ref[...]
    acc = jnp.dot(h, w1_ref[...], preferred_element_type=jnp.float32)
    for c in range(4):
        a = ss_ref[c][:, _CB:] / (ss_ref[c][:, :_CB] + 1e-16)
        acc = acc + jnp.dot(a, w1_ref[c * _CB:(c + 1) * _CB, :],
                            preferred_element_type=jnp.float32)
    inv = 1.0 / jnp.sqrt(jnp.float32(1.0 + 1e-5))
    y = (acc + b1_ref[...]) * (bng_ref[...] * inv) + bnb_ref[...]
    y = jnp.maximum(y, 0.0)
    z = jnp.dot(y, w2_ref[...], preferred_element_type=jnp.float32)
    o_ref[...] = z + b2_ref[...] + x_ref[...]


def _mlp(x, h, ss, W1, b1, bn_gamma, bn_beta, W2, b2, bn):
    n, d = x.shape
    d2 = W1.shape[1]
    return pl.pallas_call(
        _mlp_body,
        grid=(n // bn,),
        in_specs=[
            pl.BlockSpec((bn, d), lambda i: (i, 0)),
            pl.BlockSpec((bn, d), lambda i: (i, 0)),
            pl.BlockSpec((4, bn, 2 * _CB), lambda i: (0, i, 0)),
            pl.BlockSpec((d, d2), lambda i: (0, 0)),
            pl.BlockSpec((1, d2), lambda i: (0, 0)),
            pl.BlockSpec((1, d2), lambda i: (0, 0)),
            pl.BlockSpec((1, d2), lambda i: (0, 0)),
            pl.BlockSpec((d2, d), lambda i: (0, 0)),
            pl.BlockSpec((1, d), lambda i: (0, 0)),
        ],
        out_specs=pl.BlockSpec((bn, d), lambda i: (i, 0)),
        out_shape=jax.ShapeDtypeStruct((n, d), jnp.float32),
    )(x, h, ss, W1, b1.reshape(1, d2), bn_gamma.reshape(1, d2),
      bn_beta.reshape(1, d2), W2, b2.reshape(1, d))


def kernel(x, edge_index, t, W1, b1, bn_gamma, bn_beta, W2, b2,
           ln_gamma, ln_beta):
    n, d = x.shape
    e = edge_index.shape[1]
    h = _ln_relu(x, ln_gamma, ln_beta, bn=1000)
    h4 = h.reshape(n * 4, d // 4)
    tvec = jnp.full((_L,), t, dtype=jnp.float32)
    npad = -(-n // (_NS * 8)) * (_NS * 8)
    zrows = jnp.zeros((npad // _NS, 2 * _CB), dtype=jnp.float32)
    ss = _make_edge_kernel(n, e)(h4, edge_index.reshape(2 * e), tvec, zrows)
    return _mlp(x, h, ss, W1, b1, bn_gamma, bn_beta, W2, b2, bn=1000)


# speedup vs baseline: 3.8997x; 3.8997x over previous
import functools

import jax
import jax.numpy as jnp
from jax import lax
from jax.experimental import pallas as pl
from jax.experimental.pallas import tpu as pltpu
from jax.experimental.pallas import tpu_sc as plsc


def _ln_relu_body(x_ref, g_ref, b_ref, h_ref):
    x = x_ref[...]
    mu = jnp.mean(x, axis=-1, keepdims=True)
    xc = x - mu
    var = jnp.mean(xc * xc, axis=-1, keepdims=True)
    h = xc * lax.rsqrt(var + 1e-5) * g_ref[...] + b_ref[...]
    h_ref[...] = jnp.maximum(h, 0.0)


def _ln_relu(x, ln_gamma, ln_beta, bn):
    n, d = x.shape
    return pl.pallas_call(
        _ln_relu_body,
        grid=(n // bn,),
        in_specs=[
            pl.BlockSpec((bn, d), lambda i: (i, 0)),
            pl.BlockSpec((1, d), lambda i: (0, 0)),
            pl.BlockSpec((1, d), lambda i: (0, 0)),
        ],
        out_specs=pl.BlockSpec((bn, d), lambda i: (i, 0)),
        out_shape=jax.ShapeDtypeStruct((n, d), jnp.float32),
    )(x, ln_gamma.reshape(1, d), ln_beta.reshape(1, d))


_NC = 2
_NS = 16
_L = 16
_CB = 64
_PASSES = 2
_B = 80


def _make_edge_kernel(n, e):
    ept = e // _NS
    nblk = ept // _B
    npad = -(-n // (_NS * 8)) * (_NS * 8)
    rpt = npad // _NS

    assert nblk % 4 == 1 and nblk >= 5
    kmax = nblk // 4

    mesh = plsc.VectorSubcoreMesh(core_axis_name="c", subcore_axis_name="s")

    @functools.partial(
        pl.kernel,
        out_type=(
            jax.ShapeDtypeStruct((2 * _PASSES, npad, _CB), jnp.float32),
            jax.ShapeDtypeStruct((2 * _PASSES, npad, _CB), jnp.float32),
        ),
        mesh=mesh,
        scratch_types=[
            pltpu.VMEM((ept,), jnp.int32),
            [pltpu.VMEM((_B,), jnp.int32)] * 2,
            [pltpu.VMEM((_B,), jnp.int32)] * 4,
            [pltpu.VMEM((_B, _CB), jnp.float32)] * 2,
            [pltpu.VMEM((_B, _CB), jnp.float32)] * 2,
            [pltpu.VMEM((_B, _CB), jnp.float32)] * 2,
            pltpu.VMEM((_L,), jnp.float32),
            pltpu.VMEM_SHARED((npad, _CB), jnp.float32),
            pltpu.VMEM_SHARED((npad, _CB), jnp.float32),
            [pltpu.SemaphoreType.DMA] * 2,
            [pltpu.SemaphoreType.DMA] * 2,
            [pltpu.SemaphoreType.DMA] * 4,
        ],
        compiler_params=pltpu.CompilerParams(use_tc_tiling_on_sc=False),
    )
    def edge_kernel(h4, ei_flat, tvec, zrows, s1_out, s2_out,
                    src_v, gidx, sidx, rows, exb, pexb, tv,
                    acc1, acc2, gsem, ssem, dsem):
        core = lax.axis_index("c")
        sid = lax.axis_index("s")
        ebase = sid * ept
        rbase = sid * rpt

        pltpu.sync_copy(ei_flat.at[pl.ds(ebase, ept)], src_v)
        pltpu.sync_copy(tvec, tv)
        tb = tv[...]

        for cc in range(_PASSES):
            chunk = core * _PASSES + cc

            pltpu.sync_copy(zrows, acc1.at[pl.ds(rbase, rpt)])
            pltpu.sync_copy(zrows, acc2.at[pl.ds(rbase, rpt)])
            plsc.subcore_barrier()

            def build_gidx(blk, p):
                e0 = blk * _B
                for j in range(_B // _L):
                    s16 = src_v[pl.ds(e0 + j * _L, _L)]
                    gidx[p][pl.ds(j * _L, _L)] = s16 * 4 + chunk

            def compute(p):
                def edge_body(ee, c2):
                    for u in range(4):
                        for g in range(_CB // _L):
                            r = rows[p][ee * 4 + u, pl.ds(g * _L, _L)]
                            px = r + 1e-7
                            ex = jnp.exp(px * tb)
                            exb[p][ee * 4 + u, pl.ds(g * _L, _L)] = ex
                            pexb[p][ee * 4 + u, pl.ds(g * _L, _L)] = px * ex
                    return c2
                lax.fori_loop(0, _B // 4, edge_body, 0)

            def issue_gather(blk, p):
                build_gidx(blk, p)
                pltpu.async_copy(h4.at[gidx[p]], rows[p], gsem[p])

            def wait_gather(p):
                pltpu.make_async_copy(h4.at[gidx[p]], rows[p], gsem[p]).wait()

            def issue_scatter(s, p):
                pltpu.async_copy(exb[p], acc1.at[sidx[s]], ssem[p], add=True)
                pltpu.async_copy(pexb[p], acc2.at[sidx[s]], ssem[p], add=True)

            def wait_scatter(s, p):
                pltpu.make_async_copy(exb[p], acc1.at[sidx[s]], ssem[p]).wait()
                pltpu.make_async_copy(pexb[p], acc2.at[sidx[s]],
                                      ssem[p]).wait()

            def issue_dst(blk, s):
                pltpu.async_copy(
                    ei_flat.at[pl.ds(e + ebase + blk * _B, _B)], sidx[s],
                    dsem[s])

            def wait_dst(blk, s):
                pltpu.make_async_copy(
                    ei_flat.at[pl.ds(e + ebase + blk * _B, _B)], sidx[s],
                    dsem[s]).wait()

            def phase(i, k):
                blk = 4 * k + i
                p = i % 2
                snext = (i + 2) % 4

                wait_gather(p)
                if i >= 2:
                    wait_scatter(snext, p)
                else:
                    @pl.when(k > 0)
                    def _():
                        wait_scatter(snext, p)
                if i == 3:
                    @pl.when(k < kmax - 1)
                    def _():
                        issue_dst(blk + 2, snext)
                else:
                    issue_dst(blk + 2, snext)
                compute(p)
                wait_dst(blk, i)
                issue_scatter(i, p)
                if i == 3:
                    @pl.when(k < kmax - 1)
                    def _():
                        issue_gather(blk + 2, p)
                else:
                    issue_gather(blk + 2, p)

            issue_dst(0, 0)
            issue_dst(1, 1)
            issue_gather(0, 0)
            issue_gather(1, 1)

            def loop_body(k, carry):
                phase(0, k)
                phase(1, k)
                phase(2, k)
                phase(3, k)
                return carry

            lax.fori_loop(0, kmax, loop_body, 0)

            wait_gather(0)
            wait_scatter(2, 0)
            compute(0)
            wait_dst(nblk - 1, 0)
            issue_scatter(0, 0)
            wait_scatter(3, 1)
            wait_scatter(0, 0)
            plsc.subcore_barrier()

            @pl.when(core == 0)
            def _():
                pltpu.sync_copy(acc1.at[pl.ds(rbase, rpt)],
                                s1_out.at[cc, pl.ds(rbase, rpt)])
                pltpu.sync_copy(acc2.at[pl.ds(rbase, rpt)],
                                s2_out.at[cc, pl.ds(rbase, rpt)])

            @pl.when(core == 1)
            def _():
                pltpu.sync_copy(acc1.at[pl.ds(rbase, rpt)],
                                s1_out.at[_PASSES + cc, pl.ds(rbase, rpt)])
                pltpu.sync_copy(acc2.at[pl.ds(rbase, rpt)],
                                s2_out.at[_PASSES + cc, pl.ds(rbase, rpt)])

    return edge_kernel


def _mlp_body(x_ref, h_ref, s1_ref, s2_ref, w1_ref, b1_ref, bng_ref,
              bnb_ref, w2_ref, b2_ref, o_ref):
    h = h_ref[...]
    acc = jnp.dot(h, w1_ref[...], preferred_element_type=jnp.float32)
    for c in range(4):
        a = s2_ref[c] / (s1_ref[c] + 1e-16)
        acc = acc + jnp.dot(a, w1_ref[c * _CB:(c + 1) * _CB, :],
                            preferred_element_type=jnp.float32)
    inv = 1.0 / jnp.sqrt(jnp.float32(1.0 + 1e-5))
    y = (acc + b1_ref[...]) * (bng_ref[...] * inv) + bnb_ref[...]
    y = jnp.maximum(y, 0.0)
    z = jnp.dot(y, w2_ref[...], preferred_element_type=jnp.float32)
    o_ref[...] = z + b2_ref[...] + x_ref[...]


def _mlp(x, h, s1, s2, W1, b1, bn_gamma, bn_beta, W2, b2, bn):
    n, d = x.shape
    d2 = W1.shape[1]
    return pl.pallas_call(
        _mlp_body,
        grid=(n // bn,),
        in_specs=[
            pl.BlockSpec((bn, d), lambda i: (i, 0)),
            pl.BlockSpec((bn, d), lambda i: (i, 0)),
            pl.BlockSpec((4, bn, _CB), lambda i: (0, i, 0)),
            pl.BlockSpec((4, bn, _CB), lambda i: (0, i, 0)),
            pl.BlockSpec((d, d2), lambda i: (0, 0)),
            pl.BlockSpec((1, d2), lambda i: (0, 0)),
            pl.BlockSpec((1, d2), lambda i: (0, 0)),
            pl.BlockSpec((1, d2), lambda i: (0, 0)),
            pl.BlockSpec((d2, d), lambda i: (0, 0)),
            pl.BlockSpec((1, d), lambda i: (0, 0)),
        ],
        out_specs=pl.BlockSpec((bn, d), lambda i: (i, 0)),
        out_shape=jax.ShapeDtypeStruct((n, d), jnp.float32),
    )(x, h, s1, s2, W1, b1.reshape(1, d2), bn_gamma.reshape(1, d2),
      bn_beta.reshape(1, d2), W2, b2.reshape(1, d))


def kernel(x, edge_index, t, W1, b1, bn_gamma, bn_beta, W2, b2,
           ln_gamma, ln_beta):
    n, d = x.shape
    e = edge_index.shape[1]
    h = _ln_relu(x, ln_gamma, ln_beta, bn=1000)
    h4 = h.reshape(n * 4, d // 4)
    tvec = jnp.full((_L,), t, dtype=jnp.float32)
    npad = -(-n // (_NS * 8)) * (_NS * 8)
    zrows = jnp.zeros((npad // _NS, _CB), dtype=jnp.float32)
    s1, s2 = _make_edge_kernel(n, e)(h4, edge_index.reshape(2 * e),
                                     tvec, zrows)
    return _mlp(x, h, s1, s2, W1, b1, bn_gamma, bn_beta, W2, b2, bn=1000)


# speedup vs baseline: 3.9364x; 1.0094x over previous
import functools

import jax
import jax.numpy as jnp
from jax import lax
from jax.experimental import pallas as pl
from jax.experimental.pallas import tpu as pltpu
from jax.experimental.pallas import tpu_sc as plsc


def _ln_relu_body(x_ref, g_ref, b_ref, h_ref):
    x = x_ref[...]
    mu = jnp.mean(x, axis=-1, keepdims=True)
    xc = x - mu
    var = jnp.mean(xc * xc, axis=-1, keepdims=True)
    h = xc * lax.rsqrt(var + 1e-5) * g_ref[...] + b_ref[...]
    h_ref[...] = jnp.maximum(h, 0.0)


def _ln_relu(x, ln_gamma, ln_beta, bn):
    n, d = x.shape
    return pl.pallas_call(
        _ln_relu_body,
        grid=(n // bn,),
        in_specs=[
            pl.BlockSpec((bn, d), lambda i: (i, 0)),
            pl.BlockSpec((1, d), lambda i: (0, 0)),
            pl.BlockSpec((1, d), lambda i: (0, 0)),
        ],
        out_specs=pl.BlockSpec((bn, d), lambda i: (i, 0)),
        out_shape=jax.ShapeDtypeStruct((n, d), jnp.float32),
    )(x, ln_gamma.reshape(1, d), ln_beta.reshape(1, d))


_NC = 2
_NS = 16
_L = 16
_CB = 64
_PASSES = 2
_B = 80


def _make_edge_kernel(n, e):
    ept = e // _NS
    nblk = ept // _B
    npad = -(-n // (_NS * 8)) * (_NS * 8)
    rpt = npad // _NS

    assert nblk % 4 == 1 and nblk >= 5
    kmax = nblk // 4

    mesh = plsc.VectorSubcoreMesh(core_axis_name="c", subcore_axis_name="s")

    @functools.partial(
        pl.kernel,
        out_type=(
            jax.ShapeDtypeStruct((2 * _PASSES, npad, _CB), jnp.float32),
            jax.ShapeDtypeStruct((2 * _PASSES, npad, _CB), jnp.float32),
        ),
        mesh=mesh,
        scratch_types=[
            pltpu.VMEM((ept,), jnp.int32),
            [pltpu.VMEM((_B,), jnp.int32)] * 2,
            [pltpu.VMEM((_B,), jnp.int32)] * 4,
            [pltpu.VMEM((_B, _CB), jnp.float32)] * 2,
            [pltpu.VMEM((_B, _CB), jnp.float32)] * 2,
            [pltpu.VMEM((_B, _CB), jnp.float32)] * 2,
            pltpu.VMEM((_L,), jnp.float32),
            pltpu.VMEM_SHARED((npad, _CB), jnp.float32),
            pltpu.VMEM_SHARED((npad, _CB), jnp.float32),
            [pltpu.SemaphoreType.DMA] * 2,
            [pltpu.SemaphoreType.DMA] * 2,
            [pltpu.SemaphoreType.DMA] * 4,
        ],
        compiler_params=pltpu.CompilerParams(use_tc_tiling_on_sc=False),
    )
    def edge_kernel(h4, ei_flat, tvec, zrows, s1_out, s2_out,
                    src_v, gidx, sidx, rows, exb, pexb, tv,
                    acc1, acc2, gsem, ssem, dsem):
        core = lax.axis_index("c")
        sid = lax.axis_index("s")
        ebase = sid * ept
        rbase = sid * rpt

        pltpu.sync_copy(ei_flat.at[pl.ds(ebase, ept)], src_v)
        pltpu.sync_copy(tvec, tv)
        tb = tv[...]

        for cc in range(_PASSES):
            chunk = core * _PASSES + cc

            pltpu.sync_copy(zrows, acc1.at[pl.ds(rbase, rpt)])
            pltpu.sync_copy(zrows, acc2.at[pl.ds(rbase, rpt)])
            plsc.subcore_barrier()

            def build_gidx(blk, p):
                e0 = blk * _B
                for j in range(_B // _L):
                    s16 = src_v[pl.ds(e0 + j * _L, _L)]
                    gidx[p][pl.ds(j * _L, _L)] = s16 * 4 + chunk

            def compute(p):
                def edge_body(ee, c2):
                    for u in range(4):
                        for g in range(_CB // _L):
                            r = rows[p][ee * 4 + u, pl.ds(g * _L, _L)]
                            px = r + 1e-7
                            ex = jnp.exp(px * tb)
                            exb[p][ee * 4 + u, pl.ds(g * _L, _L)] = ex
                            pexb[p][ee * 4 + u, pl.ds(g * _L, _L)] = px * ex
                    return c2
                lax.fori_loop(0, _B // 4, edge_body, 0)

            def issue_gather(blk, p):
                build_gidx(blk, p)
                pltpu.async_copy(h4.at[gidx[p]], rows[p], gsem[p])

            def wait_gather(p):
                pltpu.make_async_copy(h4.at[gidx[p]], rows[p], gsem[p]).wait()

            def issue_scatter(s, p):
                pltpu.async_copy(exb[p], acc1.at[sidx[s]], ssem[p], add=True)
                pltpu.async_copy(pexb[p], acc2.at[sidx[s]], ssem[p], add=True)

            def wait_scatter(s, p):
                pltpu.make_async_copy(exb[p], acc1.at[sidx[s]], ssem[p]).wait()
                pltpu.make_async_copy(pexb[p], acc2.at[sidx[s]],
                                      ssem[p]).wait()

            def issue_dst(blk, s):
                pltpu.async_copy(
                    ei_flat.at[pl.ds(e + ebase + blk * _B, _B)], sidx[s],
                    dsem[s])

            def wait_dst(blk, s):
                pltpu.make_async_copy(
                    ei_flat.at[pl.ds(e + ebase + blk * _B, _B)], sidx[s],
                    dsem[s]).wait()

            def phase(i, k):
                blk = 4 * k + i
                p = i % 2
                snext = (i + 2) % 4

                wait_gather(p)
                if i >= 2:
                    wait_scatter(snext, p)
                else:
                    @pl.when(k > 0)
                    def _():
                        wait_scatter(snext, p)
                if i == 3:
                    @pl.when(k < kmax - 1)
                    def _():
                        issue_dst(blk + 2, snext)
                else:
                    issue_dst(blk + 2, snext)
                compute(p)
                wait_dst(blk, i)
                issue_scatter(i, p)
                if i == 3:
                    @pl.when(k < kmax - 1)
                    def _():
                        issue_gather(blk + 2, p)
                else:
                    issue_gather(blk + 2, p)

            issue_dst(0, 0)
            issue_dst(1, 1)
            issue_gather(0, 0)
            issue_gather(1, 1)

            def loop_body(k, carry):
                phase(0, k)
                phase(1, k)
                phase(2, k)
                phase(3, k)
                return carry

            lax.fori_loop(0, kmax, loop_body, 0)

            wait_gather(0)
            wait_scatter(2, 0)
            compute(0)
            wait_dst(nblk - 1, 0)
            issue_scatter(0, 0)
            wait_scatter(3, 1)
            wait_scatter(0, 0)
            plsc.subcore_barrier()

            @pl.when(core == 0)
            def _():
                pltpu.sync_copy(acc1.at[pl.ds(rbase, rpt)],
                                s1_out.at[cc, pl.ds(rbase, rpt)])
                pltpu.sync_copy(acc2.at[pl.ds(rbase, rpt)],
                                s2_out.at[cc, pl.ds(rbase, rpt)])

            @pl.when(core == 1)
            def _():
                pltpu.sync_copy(acc1.at[pl.ds(rbase, rpt)],
                                s1_out.at[_PASSES + cc, pl.ds(rbase, rpt)])
                pltpu.sync_copy(acc2.at[pl.ds(rbase, rpt)],
                                s2_out.at[_PASSES + cc, pl.ds(rbase, rpt)])

    return edge_kernel


def _mlp_body(x_ref, h_ref, s1_ref, s2_ref, w1_ref, b1_ref, bng_ref,
              bnb_ref, w2_ref, b2_ref, o_ref):
    h = h_ref[...]
    acc = jnp.dot(h, w1_ref[...], preferred_element_type=jnp.float32)
    for c in range(4):
        a = s2_ref[c] / (s1_ref[c] + 1e-16)
        acc = acc + jnp.dot(a, w1_ref[c * _CB:(c + 1) * _CB, :],
                            preferred_element_type=jnp.float32)
    inv = 1.0 / jnp.sqrt(jnp.float32(1.0 + 1e-5))
    y = (acc + b1_ref[...]) * (bng_ref[...] * inv) + bnb_ref[...]
    y = jnp.maximum(y, 0.0)
    z = jnp.dot(y, w2_ref[...], preferred_element_type=jnp.float32)
    o_ref[...] = z + b2_ref[...] + x_ref[...]


def _mlp(x, h, s1, s2, W1, b1, bn_gamma, bn_beta, W2, b2, bn):
    n, d = x.shape
    d2 = W1.shape[1]
    return pl.pallas_call(
        _mlp_body,
        grid=(n // bn,),
        in_specs=[
            pl.BlockSpec((bn, d), lambda i: (i, 0)),
            pl.BlockSpec((bn, d), lambda i: (i, 0)),
            pl.BlockSpec((4, bn, _CB), lambda i: (0, i, 0)),
            pl.BlockSpec((4, bn, _CB), lambda i: (0, i, 0)),
            pl.BlockSpec((d, d2), lambda i: (0, 0)),
            pl.BlockSpec((1, d2), lambda i: (0, 0)),
            pl.BlockSpec((1, d2), lambda i: (0, 0)),
            pl.BlockSpec((1, d2), lambda i: (0, 0)),
            pl.BlockSpec((d2, d), lambda i: (0, 0)),
            pl.BlockSpec((1, d), lambda i: (0, 0)),
        ],
        out_specs=pl.BlockSpec((bn, d), lambda i: (i, 0)),
        out_shape=jax.ShapeDtypeStruct((n, d), jnp.float32),
    )(x, h, s1, s2, W1, b1.reshape(1, d2), bn_gamma.reshape(1, d2),
      bn_beta.reshape(1, d2), W2, b2.reshape(1, d))


def kernel(x, edge_index, t, W1, b1, bn_gamma, bn_beta, W2, b2,
           ln_gamma, ln_beta):
    n, d = x.shape
    e = edge_index.shape[1]
    h = _ln_relu(x, ln_gamma, ln_beta, bn=2000)
    h4 = h.reshape(n * 4, d // 4)
    tvec = jnp.full((_L,), t, dtype=jnp.float32)
    npad = -(-n // (_NS * 8)) * (_NS * 8)
    zrows = jnp.zeros((npad // _NS, _CB), dtype=jnp.float32)
    s1, s2 = _make_edge_kernel(n, e)(h4, edge_index.reshape(2 * e),
                                     tvec, zrows)
    return _mlp(x, h, s1, s2, W1, b1, bn_gamma, bn_beta, W2, b2,
                bn=2000)
